# baseline (speedup 1.0000x reference)
import functools

import jax
import jax.numpy as jnp
from jax import lax
from jax.experimental import pallas as pl
from jax.experimental.pallas import tpu as pltpu
from jax.experimental.pallas import tpu_sc as plsc

_N = 4096
_L = 16
_NC = 2
_NS = 16
_NW = _NC * _NS
_RPW = _N // _NW
_JV = _N // _L
_ALPHA = 4.0

_mesh = plsc.VectorSubcoreMesh(core_axis_name="c", subcore_axis_name="s")

_GATHER_DNUMS = lax.GatherDimensionNumbers(
    offset_dims=(), collapsed_slice_dims=(0,), start_index_map=(0,))


def _bcast_lane(vec, k):
    kidx = jnp.full((_L,), k, jnp.int32)
    return lax.gather(vec, kidx[:, None], _GATHER_DNUMS, slice_sizes=(1,),
                      mode=lax.GatherScatterMode.PROMISE_IN_BOUNDS)


@functools.partial(
    pl.kernel,
    out_type=jax.ShapeDtypeStruct((_NW, 3, _L), jnp.float32),
    mesh=_mesh,
    scratch_types=[
        pltpu.VMEM((_N,), jnp.float32),
        pltpu.VMEM((_N,), jnp.float32),
        pltpu.VMEM((3, _L), jnp.float32),
    ],
)
def _sc_loss(p_hbm, t_hbm, out_hbm, p_v, t_v, acc_v):
    c = lax.axis_index("c")
    s = lax.axis_index("s")
    wid = s * _NC + c
    row0 = wid * _RPW

    pltpu.sync_copy(p_hbm, p_v)
    pltpu.sync_copy(t_hbm, t_v)

    zero = jnp.zeros((_L,), jnp.float32)

    def group_body(g, carry):
        pg = p_v[pl.ds(row0 + g * _L, _L)]
        tg = t_v[pl.ds(row0 + g * _L, _L)]

        def lane_body(k, kcarry):
            pib = _bcast_lane(pg, k)
            tib = _bcast_lane(tg, k)

            def j_body(j, jcarry):
                a_s, a_c = jcarry
                pj = p_v[pl.ds(j * _L, _L)]
                tj = t_v[pl.ds(j * _L, _L)]
                dt = tib - tj
                sg = jnp.sign(dt)
                m = sg * sg
                cc = jnp.maximum(m - (pib - pj) * sg, 0.0)
                return (a_s + cc, a_c + m)

            return lax.fori_loop(0, _JV, j_body, kcarry)

        return lax.fori_loop(0, _L, lane_body, carry)

    acc_s, acc_c = lax.fori_loop(0, _RPW // _L, group_body, (zero, zero))

    def mse_body(g, a_e):
        e = p_v[pl.ds(row0 + g * _L, _L)] - t_v[pl.ds(row0 + g * _L, _L)]
        return a_e + e * e

    acc_e = lax.fori_loop(0, _RPW // _L, mse_body, zero)

    acc_v[0, :] = acc_s
    acc_v[1, :] = acc_c
    acc_v[2, :] = acc_e

    pltpu.sync_copy(acc_v, out_hbm.at[wid])


@jax.jit
def kernel(pred, target):
    p = pred.reshape(_N)
    t = target.reshape(_N)
    parts = _sc_loss(p, t)
    pair_sum = jnp.sum(parts[:, 0, :])
    pair_cnt = jnp.sum(parts[:, 1, :])
    sq_err = jnp.sum(parts[:, 2, :])
    return sq_err / _N + _ALPHA * pair_sum / jnp.maximum(pair_cnt, 1.0)


# speedup vs baseline: 1.8037x; 1.8037x over previous
import functools

import jax
import jax.numpy as jnp
from jax import lax
from jax.experimental import pallas as pl
from jax.experimental.pallas import tpu as pltpu
from jax.experimental.pallas import tpu_sc as plsc

_N = 4096
_L = 16
_NC = 2
_NS = 16
_NW = _NC * _NS
_NG = _N // _L
_GPW = _NG // _NW
_ALPHA = 4.0

_mesh = plsc.VectorSubcoreMesh(core_axis_name="c", subcore_axis_name="s")

_GATHER_DNUMS = lax.GatherDimensionNumbers(
    offset_dims=(), collapsed_slice_dims=(0,), start_index_map=(0,))


def _bcast_lane(vec, k):
    kidx = jnp.full((_L,), k, jnp.int32)
    return lax.gather(vec, kidx[:, None], _GATHER_DNUMS, slice_sizes=(1,),
                      mode=lax.GatherScatterMode.PROMISE_IN_BOUNDS)


@functools.partial(
    pl.kernel,
    out_type=jax.ShapeDtypeStruct((_NW, 3, _L), jnp.float32),
    mesh=_mesh,
    scratch_types=[
        pltpu.VMEM((_N,), jnp.float32),
        pltpu.VMEM((_N,), jnp.float32),
        pltpu.VMEM((3, _L), jnp.float32),
    ],
)
def _sc_loss(p_hbm, t_hbm, out_hbm, p_v, t_v, acc_v):
    c = lax.axis_index("c")
    s = lax.axis_index("s")
    wid = s * _NC + c

    pltpu.sync_copy(p_hbm, p_v)
    pltpu.sync_copy(t_hbm, t_v)

    zero = jnp.zeros((_L,), jnp.float32)
    lane = lax.iota(jnp.int32, _L)

    def group_body(q, carry):
        acc_s, acc_c, acc_e = carry
        grp = wid + q * _NW
        base = grp * _L
        pg = p_v[pl.ds(base, _L)]
        tg = t_v[pl.ds(base, _L)]

        e = pg - tg
        acc_e = acc_e + e * e

        pib = [_bcast_lane(pg, k) for k in range(_L)]
        tib = [_bcast_lane(tg, k) for k in range(_L)]

        for k in range(_L):
            dt = tib[k] - tg
            sg = jnp.sign(dt)
            m = jnp.where(lane > k, sg * sg, 0.0)
            cc = jnp.maximum(m - (pib[k] - pg) * (jnp.where(lane > k, sg, 0.0)), 0.0)
            acc_s = acc_s + cc
            acc_c = acc_c + m

        def j_body(jv, jcarry):
            a_s, a_c = jcarry
            pj = p_v[pl.ds(jv * _L, _L)]
            tj = t_v[pl.ds(jv * _L, _L)]
            for k in range(_L):
                dt = tib[k] - tj
                sg = jnp.sign(dt)
                m = sg * sg
                cc = jnp.maximum(m - (pib[k] - pj) * sg, 0.0)
                a_s = a_s + cc
                a_c = a_c + m
            return (a_s, a_c)

        acc_s, acc_c = lax.fori_loop(grp + 1, _NG, j_body, (acc_s, acc_c))
        return (acc_s, acc_c, acc_e)

    acc_s, acc_c, acc_e = lax.fori_loop(0, _GPW, group_body, (zero, zero, zero))

    acc_v[0, :] = acc_s
    acc_v[1, :] = acc_c
    acc_v[2, :] = acc_e

    pltpu.sync_copy(acc_v, out_hbm.at[wid])


@jax.jit
def kernel(pred, target):
    p = pred.reshape(_N)
    t = target.reshape(_N)
    parts = _sc_loss(p, t)
    pair_sum = jnp.sum(parts[:, 0, :])
    pair_cnt = jnp.sum(parts[:, 1, :])
    sq_err = jnp.sum(parts[:, 2, :])
    return sq_err / _N + _ALPHA * pair_sum / jnp.maximum(pair_cnt, 1.0)


# speedup vs baseline: 2.4957x; 1.3837x over previous
import functools

import jax
import jax.numpy as jnp
from jax import lax
from jax.experimental import pallas as pl
from jax.experimental.pallas import tpu as pltpu
from jax.experimental.pallas import tpu_sc as plsc

_N = 4096
_ALPHA = 4.0


_R = 2048
_B = 512


_L = 16
_NC = 2
_NS = 16
_NW = _NC * _NS
_NG = _N // _L
_G0 = _R // _L
_GPW = (_NG - _G0) // _NW

_mesh = plsc.VectorSubcoreMesh(core_axis_name="c", subcore_axis_name="s")

_GATHER_DNUMS = lax.GatherDimensionNumbers(
    offset_dims=(), collapsed_slice_dims=(0,), start_index_map=(0,))


def _bcast_lane(vec, k):
    kidx = jnp.full((_L,), k, jnp.int32)
    return lax.gather(vec, kidx[:, None], _GATHER_DNUMS, slice_sizes=(1,),
                      mode=lax.GatherScatterMode.PROMISE_IN_BOUNDS)


def _tc_band(pc_ref, tc_ref, pr_ref, tr_ref, out_ref, acc_ref):
    i = pl.program_id(0)

    pi = pc_ref[...]
    ti = tc_ref[...]
    pj = pr_ref[...]
    tj = tr_ref[...]

    row_id = i * _B + lax.broadcasted_iota(jnp.int32, (_B, 1), 0)
    col_id = lax.broadcasted_iota(jnp.int32, (1, _N), 1)
    mf = jnp.where(col_id > row_id, 1.0, 0.0)

    dt = ti - tj
    s = jnp.sign(dt) * mf
    m = s * s
    dp = pi - pj
    cc = jnp.maximum(m - dp * s, 0.0)

    s_part = jnp.sum(cc)
    c_part = jnp.sum(m)
    e = pi - ti
    mse_part = jnp.sum(e * e)

    @pl.when(i == 0)
    def _init():
        acc_ref[0] = 0.0
        acc_ref[1] = 0.0
        acc_ref[2] = 0.0

    acc_ref[0] += s_part
    acc_ref[1] += c_part
    acc_ref[2] += mse_part

    @pl.when(i == pl.num_programs(0) - 1)
    def _finish():
        out_ref[0] = acc_ref[0]
        out_ref[1] = acc_ref[1]
        out_ref[2] = acc_ref[2]


def _tc_call(pc, tc_, pr, tr):
    return pl.pallas_call(
        _tc_band,
        grid=(_R // _B,),
        in_specs=[
            pl.BlockSpec((_B, 1), lambda i: (i, 0)),
            pl.BlockSpec((_B, 1), lambda i: (i, 0)),
            pl.BlockSpec((1, _N), lambda i: (0, 0)),
            pl.BlockSpec((1, _N), lambda i: (0, 0)),
        ],
        out_specs=pl.BlockSpec(memory_space=pltpu.SMEM),
        out_shape=jax.ShapeDtypeStruct((4,), jnp.float32),
        scratch_shapes=[pltpu.SMEM((4,), jnp.float32)],
    )(pc, tc_, pr, tr)


@functools.partial(
    pl.kernel,
    out_type=jax.ShapeDtypeStruct((_NW, 3, _L), jnp.float32),
    mesh=_mesh,
    scratch_types=[
        pltpu.VMEM((_N,), jnp.float32),
        pltpu.VMEM((_N,), jnp.float32),
        pltpu.VMEM((3, _L), jnp.float32),
    ],
)
def _sc_loss(p_hbm, t_hbm, out_hbm, p_v, t_v, acc_v):
    c = lax.axis_index("c")
    s = lax.axis_index("s")
    wid = s * _NC + c

    pltpu.sync_copy(p_hbm, p_v)
    pltpu.sync_copy(t_hbm, t_v)

    zero = jnp.zeros((_L,), jnp.float32)
    lane = lax.iota(jnp.int32, _L)

    def group_body(q, carry):
        acc_s, acc_c, acc_e = carry
        grp = _G0 + wid + q * _NW
        base = grp * _L
        pg = p_v[pl.ds(base, _L)]
        tg = t_v[pl.ds(base, _L)]

        e = pg - tg
        acc_e = acc_e + e * e

        pib = [_bcast_lane(pg, k) for k in range(_L)]
        tib = [_bcast_lane(tg, k) for k in range(_L)]

        for k in range(_L):
            dt = tib[k] - tg
            sg = jnp.where(lane > k, jnp.sign(dt), 0.0)
            m = sg * sg
            cc = jnp.maximum(m - (pib[k] - pg) * sg, 0.0)
            acc_s = acc_s + cc
            acc_c = acc_c + m

        def j_body(jv, jcarry):
            a_s, a_c = jcarry
            pj = p_v[pl.ds(jv * _L, _L)]
            tj = t_v[pl.ds(jv * _L, _L)]
            for k in range(_L):
                dt = tib[k] - tj
                sg = jnp.sign(dt)
                m = sg * sg
                cc = jnp.maximum(m - (pib[k] - pj) * sg, 0.0)
                a_s = a_s + cc
                a_c = a_c + m
            return (a_s, a_c)

        acc_s, acc_c = lax.fori_loop(grp + 1, _NG, j_body, (acc_s, acc_c))
        return (acc_s, acc_c, acc_e)

    acc_s, acc_c, acc_e = lax.fori_loop(0, _GPW, group_body, (zero, zero, zero))

    acc_v[0, :] = acc_s
    acc_v[1, :] = acc_c
    acc_v[2, :] = acc_e

    pltpu.sync_copy(acc_v, out_hbm.at[wid])


@jax.jit
def kernel(pred, target):
    p = pred.reshape(_N)
    t = target.reshape(_N)
    sc_parts = _sc_loss(p, t)
    tc_parts = _tc_call(pred.reshape(_N, 1), target.reshape(_N, 1),
                        pred.reshape(1, _N), target.reshape(1, _N))
    pair_sum = jnp.sum(sc_parts[:, 0, :]) + tc_parts[0]
    pair_cnt = jnp.sum(sc_parts[:, 1, :]) + tc_parts[1]
    sq_err = jnp.sum(sc_parts[:, 2, :]) + tc_parts[2]
    return sq_err / _N + _ALPHA * pair_sum / jnp.maximum(pair_cnt, 1.0)


# speedup vs baseline: 2.7081x; 1.0851x over previous
import functools

import jax
import jax.numpy as jnp
from jax import lax
from jax.experimental import pallas as pl
from jax.experimental.pallas import tpu as pltpu
from jax.experimental.pallas import tpu_sc as plsc

_N = 4096
_ALPHA = 4.0


_R = 2048
_B = 512


_L = 16
_NC = 2
_NS = 16
_NW = _NC * _NS
_NG = _N // _L
_G0 = _R // _L
_GPW = (_NG - _G0) // _NW

_mesh = plsc.VectorSubcoreMesh(core_axis_name="c", subcore_axis_name="s")

_GATHER_DNUMS = lax.GatherDimensionNumbers(
    offset_dims=(), collapsed_slice_dims=(0,), start_index_map=(0,))


def _bcast_lane(vec, k):
    kidx = jnp.full((_L,), k, jnp.int32)
    return lax.gather(vec, kidx[:, None], _GATHER_DNUMS, slice_sizes=(1,),
                      mode=lax.GatherScatterMode.PROMISE_IN_BOUNDS)


def _tc_band(pc_ref, tc_ref, pr_ref, tr_ref, out_ref, acc_ref):
    ib = pl.program_id(0)
    jb = pl.program_id(1)

    @pl.when(jnp.logical_and(ib == 0, jb == 0))
    def _init():
        acc_ref[0] = 0.0
        acc_ref[1] = 0.0
        acc_ref[2] = 0.0

    @pl.when(jb > ib)
    def _full_block():
        pi = pc_ref[...]
        ti = tc_ref[...]
        pj = pr_ref[...]
        tj = tr_ref[...]
        dt = ti - tj
        s = jnp.sign(dt)
        m = s * s
        cc = jnp.maximum(m - (pi - pj) * s, 0.0)
        acc_ref[0] += jnp.sum(cc)
        acc_ref[1] += jnp.sum(m)

    @pl.when(jb == ib)
    def _diag_block():
        pi = pc_ref[...]
        ti = tc_ref[...]
        pj = pr_ref[...]
        tj = tr_ref[...]
        row_id = lax.broadcasted_iota(jnp.int32, (_B, 1), 0)
        col_id = lax.broadcasted_iota(jnp.int32, (1, _B), 1)
        mf = jnp.where(col_id > row_id, 1.0, 0.0)
        dt = ti - tj
        s = jnp.sign(dt) * mf
        m = s * s
        cc = jnp.maximum(m - (pi - pj) * s, 0.0)
        e = pi - ti
        acc_ref[0] += jnp.sum(cc)
        acc_ref[1] += jnp.sum(m)
        acc_ref[2] += jnp.sum(e * e)

    @pl.when(jnp.logical_and(ib == _R // _B - 1, jb == _N // _B - 1))
    def _finish():
        out_ref[0] = acc_ref[0]
        out_ref[1] = acc_ref[1]
        out_ref[2] = acc_ref[2]


def _tc_call(pc, tc_, pr, tr):
    return pl.pallas_call(
        _tc_band,
        grid=(_R // _B, _N // _B),
        in_specs=[
            pl.BlockSpec((_B, 1), lambda i, j: (i, 0)),
            pl.BlockSpec((_B, 1), lambda i, j: (i, 0)),
            pl.BlockSpec((1, _B), lambda i, j: (0, j)),
            pl.BlockSpec((1, _B), lambda i, j: (0, j)),
        ],
        out_specs=pl.BlockSpec(memory_space=pltpu.SMEM),
        out_shape=jax.ShapeDtypeStruct((4,), jnp.float32),
        scratch_shapes=[pltpu.SMEM((4,), jnp.float32)],
    )(pc, tc_, pr, tr)


@functools.partial(
    pl.kernel,
    out_type=jax.ShapeDtypeStruct((_NW, 3, _L), jnp.float32),
    mesh=_mesh,
    scratch_types=[
        pltpu.VMEM((_N,), jnp.float32),
        pltpu.VMEM((_N,), jnp.float32),
        pltpu.VMEM((3, _L), jnp.float32),
    ],
)
def _sc_loss(p_hbm, t_hbm, out_hbm, p_v, t_v, acc_v):
    c = lax.axis_index("c")
    s = lax.axis_index("s")
    wid = s * _NC + c

    pltpu.sync_copy(p_hbm, p_v)
    pltpu.sync_copy(t_hbm, t_v)

    zero = jnp.zeros((_L,), jnp.float32)
    lane = lax.iota(jnp.int32, _L)

    def group_body(q, carry):
        acc_s, acc_c, acc_e = carry
        grp = _G0 + wid + q * _NW
        base = grp * _L
        pg = p_v[pl.ds(base, _L)]
        tg = t_v[pl.ds(base, _L)]

        e = pg - tg
        acc_e = acc_e + e * e

        pib = [_bcast_lane(pg, k) for k in range(_L)]
        tib = [_bcast_lane(tg, k) for k in range(_L)]

        for k in range(_L):
            dt = tib[k] - tg
            sg = jnp.where(lane > k, jnp.sign(dt), 0.0)
            m = sg * sg
            cc = jnp.maximum(m - (pib[k] - pg) * sg, 0.0)
            acc_s = acc_s + cc
            acc_c = acc_c + m

        def j_body(jv, jcarry):
            a_s, a_c = jcarry
            pj = p_v[pl.ds(jv * _L, _L)]
            tj = t_v[pl.ds(jv * _L, _L)]
            for k in range(_L):
                dt = tib[k] - tj
                sg = jnp.sign(dt)
                m = sg * sg
                cc = jnp.maximum(m - (pib[k] - pj) * sg, 0.0)
                a_s = a_s + cc
                a_c = a_c + m
            return (a_s, a_c)

        acc_s, acc_c = lax.fori_loop(grp + 1, _NG, j_body, (acc_s, acc_c))
        return (acc_s, acc_c, acc_e)

    acc_s, acc_c, acc_e = lax.fori_loop(0, _GPW, group_body, (zero, zero, zero))

    acc_v[0, :] = acc_s
    acc_v[1, :] = acc_c
    acc_v[2, :] = acc_e

    pltpu.sync_copy(acc_v, out_hbm.at[wid])


@jax.jit
def kernel(pred, target):
    p = pred.reshape(_N)
    t = target.reshape(_N)
    sc_parts = _sc_loss(p, t)
    tc_parts = _tc_call(pred.reshape(_N, 1), target.reshape(_N, 1),
                        pred.reshape(1, _N), target.reshape(1, _N))
    pair_sum = jnp.sum(sc_parts[:, 0, :]) + tc_parts[0]
    pair_cnt = jnp.sum(sc_parts[:, 1, :]) + tc_parts[1]
    sq_err = jnp.sum(sc_parts[:, 2, :]) + tc_parts[2]
    return sq_err / _N + _ALPHA * pair_sum / jnp.maximum(pair_cnt, 1.0)


# speedup vs baseline: 2.7489x; 1.0151x over previous
import functools

import jax
import jax.numpy as jnp
from jax import lax
from jax.experimental import pallas as pl
from jax.experimental.pallas import tpu as pltpu
from jax.experimental.pallas import tpu_sc as plsc

_N = 4096
_ALPHA = 4.0


_R = 2048
_B = 512


_L = 16
_NC = 2
_NS = 16
_NW = _NC * _NS
_NG = _N // _L
_G0 = _R // _L
_GPW = (_NG - _G0) // _NW

_mesh = plsc.VectorSubcoreMesh(core_axis_name="c", subcore_axis_name="s")

_GATHER_DNUMS = lax.GatherDimensionNumbers(
    offset_dims=(), collapsed_slice_dims=(0,), start_index_map=(0,))


def _bcast_lane(vec, k):
    kidx = jnp.full((_L,), k, jnp.int32)
    return lax.gather(vec, kidx[:, None], _GATHER_DNUMS, slice_sizes=(1,),
                      mode=lax.GatherScatterMode.PROMISE_IN_BOUNDS)


def _tc_band(pc_ref, tc_ref, pr_ref, tr_ref, out_ref, vs_ref, vc_ref, sm_ref):
    ib = pl.program_id(0)
    jb = pl.program_id(1)

    @pl.when(jnp.logical_and(ib == 0, jb == 0))
    def _init():
        zrow = jnp.zeros((1, _B), jnp.float32)
        vs_ref[...] = zrow
        vc_ref[...] = zrow
        sm_ref[0] = 0.0

    @pl.when(jb > ib)
    def _full_block():
        pi = pc_ref[...]
        ti = tc_ref[...]
        pj = pr_ref[...]
        tj = tr_ref[...]
        dt = ti - tj
        s = jnp.sign(dt)
        m = s * s
        cc = jnp.maximum(m - (pi - pj) * s, 0.0)
        vs_ref[...] += jnp.sum(cc, axis=0, keepdims=True)
        vc_ref[...] += jnp.sum(m, axis=0, keepdims=True)

    @pl.when(jb == ib)
    def _diag_block():
        pi = pc_ref[...]
        ti = tc_ref[...]
        pj = pr_ref[...]
        tj = tr_ref[...]
        row_id = lax.broadcasted_iota(jnp.int32, (_B, 1), 0)
        col_id = lax.broadcasted_iota(jnp.int32, (1, _B), 1)
        mf = jnp.where(col_id > row_id, 1.0, 0.0)
        dt = ti - tj
        s = jnp.sign(dt) * mf
        m = s * s
        cc = jnp.maximum(m - (pi - pj) * s, 0.0)
        e = pi - ti
        vs_ref[...] += jnp.sum(cc, axis=0, keepdims=True)
        vc_ref[...] += jnp.sum(m, axis=0, keepdims=True)
        sm_ref[0] += jnp.sum(e * e)

    @pl.when(jnp.logical_and(ib == _R // _B - 1, jb == _N // _B - 1))
    def _finish():
        out_ref[0] = jnp.sum(vs_ref[...])
        out_ref[1] = jnp.sum(vc_ref[...])
        out_ref[2] = sm_ref[0]


def _tc_call(pc, tc_, pr, tr):
    return pl.pallas_call(
        _tc_band,
        grid=(_R // _B, _N // _B),
        in_specs=[
            pl.BlockSpec((_B, 1), lambda i, j: (i, 0)),
            pl.BlockSpec((_B, 1), lambda i, j: (i, 0)),
            pl.BlockSpec((1, _B), lambda i, j: (0, j)),
            pl.BlockSpec((1, _B), lambda i, j: (0, j)),
        ],
        out_specs=pl.BlockSpec(memory_space=pltpu.SMEM),
        out_shape=jax.ShapeDtypeStruct((4,), jnp.float32),
        scratch_shapes=[
            pltpu.VMEM((1, _B), jnp.float32),
            pltpu.VMEM((1, _B), jnp.float32),
            pltpu.SMEM((4,), jnp.float32),
        ],
    )(pc, tc_, pr, tr)


@functools.partial(
    pl.kernel,
    out_type=jax.ShapeDtypeStruct((_NW, 3, _L), jnp.float32),
    mesh=_mesh,
    scratch_types=[
        pltpu.VMEM((_N,), jnp.float32),
        pltpu.VMEM((_N,), jnp.float32),
        pltpu.VMEM((3, _L), jnp.float32),
    ],
)
def _sc_loss(p_hbm, t_hbm, out_hbm, p_v, t_v, acc_v):
    c = lax.axis_index("c")
    s = lax.axis_index("s")
    wid = s * _NC + c

    pltpu.sync_copy(p_hbm, p_v)
    pltpu.sync_copy(t_hbm, t_v)

    zero = jnp.zeros((_L,), jnp.float32)
    lane = lax.iota(jnp.int32, _L)

    def group_body(q, carry):
        acc_s, acc_c, acc_e = carry
        grp = _G0 + wid + q * _NW
        base = grp * _L
        pg = p_v[pl.ds(base, _L)]
        tg = t_v[pl.ds(base, _L)]

        e = pg - tg
        acc_e = acc_e + e * e

        pib = [_bcast_lane(pg, k) for k in range(_L)]
        tib = [_bcast_lane(tg, k) for k in range(_L)]

        for k in range(_L):
            dt = tib[k] - tg
            sg = jnp.where(lane > k, jnp.sign(dt), 0.0)
            m = sg * sg
            cc = jnp.maximum(m - (pib[k] - pg) * sg, 0.0)
            acc_s = acc_s + cc
            acc_c = acc_c + m

        def j_body(jv, jcarry):
            a_s, a_c = jcarry
            pj = p_v[pl.ds(jv * _L, _L)]
            tj = t_v[pl.ds(jv * _L, _L)]
            for k in range(_L):
                dt = tib[k] - tj
                sg = jnp.sign(dt)
                m = sg * sg
                cc = jnp.maximum(m - (pib[k] - pj) * sg, 0.0)
                a_s = a_s + cc
                a_c = a_c + m
            return (a_s, a_c)

        acc_s, acc_c = lax.fori_loop(grp + 1, _NG, j_body, (acc_s, acc_c))
        return (acc_s, acc_c, acc_e)

    acc_s, acc_c, acc_e = lax.fori_loop(0, _GPW, group_body, (zero, zero, zero))

    acc_v[0, :] = acc_s
    acc_v[1, :] = acc_c
    acc_v[2, :] = acc_e

    pltpu.sync_copy(acc_v, out_hbm.at[wid])


@jax.jit
def kernel(pred, target):
    p = pred.reshape(_N)
    t = target.reshape(_N)
    sc_parts = _sc_loss(p, t)
    tc_parts = _tc_call(pred.reshape(_N, 1), target.reshape(_N, 1),
                        pred.reshape(1, _N), target.reshape(1, _N))
    pair_sum = jnp.sum(sc_parts[:, 0, :]) + tc_parts[0]
    pair_cnt = jnp.sum(sc_parts[:, 1, :]) + tc_parts[1]
    sq_err = jnp.sum(sc_parts[:, 2, :]) + tc_parts[2]
    return sq_err / _N + _ALPHA * pair_sum / jnp.maximum(pair_cnt, 1.0)
